# baseline (speedup 1.0000x reference)
import functools

import jax
import jax.numpy as jnp
from jax.experimental import pallas as pl
import jax.experimental.pallas.tpu as pltpu


def _body(xt_ref, yt_ref, a_ref, b_ref, vkyw_ref, vkyb_ref, wkxw_ref,
          wkxb_ref, vkxw_ref, vkxb_ref, sig_ref, out_ref, u_ref, vky_ref):
    i = pl.program_id(0)
    bm = a_ref.shape[0]

    @pl.when(i == 0)
    def _prologue():
        cw = vkxw_ref[...] - 2.0 * wkxw_ref[...]
        cb = vkxb_ref[...] - 2.0 * wkxb_ref[...]
        u_ref[...] = (
            jax.lax.dot_general(
                xt_ref[...], cw,
                (((0,), (1,)), ((), ())),
                preferred_element_type=jnp.float32,
            )
            + cb[None, :]
        ).astype(jnp.bfloat16)
        vky_ref[...] = (
            jnp.dot(vkyw_ref[...], yt_ref[...],
                    preferred_element_type=jnp.float32)
            + vkyb_ref[...][:, None]
        )

    t = jnp.dot(
        a_ref[...].astype(jnp.bfloat16),
        u_ref[...],
        preferred_element_type=jnp.float32,
    )
    tt = t.T
    vky_sl = vky_ref[:, pl.ds(i * bm, bm)]
    b_sl = b_ref[pl.ds(i * bm, bm)]
    out_ref[...] = jnp.maximum(
        vky_sl - sig_ref[0] * (b_sl[None, :] + tt), 0.0
    )


@functools.partial(jax.jit, static_argnames=())
def kernel(x, y, A, b, Vky_W, Vky_b, Wkx_W, Wkx_b, Vkx_W, Vkx_b, sigma):
    n, d = x.shape
    bm = 256
    grid = (n // bm,)

    full = lambda shape: pl.BlockSpec(shape, lambda i: (0, 0))
    anyb = lambda: pl.BlockSpec(memory_space=pltpu.VMEM)

    out_t = pl.pallas_call(
        _body,
        grid=grid,
        in_specs=[
            full((d, n)),
            full((d, n)),
            pl.BlockSpec((bm, n), lambda i: (i, 0)),
            anyb(),
            full((d, d)),
            anyb(),
            full((d, d)),
            anyb(),
            full((d, d)),
            anyb(),
            pl.BlockSpec(memory_space=pltpu.SMEM),
        ],
        out_specs=pl.BlockSpec((d, bm), lambda i: (0, i)),
        out_shape=jax.ShapeDtypeStruct((d, n), jnp.float32),
        scratch_shapes=[
            pltpu.VMEM((n, d), jnp.bfloat16),
            pltpu.VMEM((d, n), jnp.float32),
        ],
    )(
        x.T, y.T, A, b.reshape(n),
        Vky_W, Vky_b,
        Wkx_W, Wkx_b,
        Vkx_W, Vkx_b,
        sigma,
    )
    return out_t.T


# speedup vs baseline: 1.1439x; 1.1439x over previous
import functools

import jax
import jax.numpy as jnp
from jax.experimental import pallas as pl
import jax.experimental.pallas.tpu as pltpu


def _body(xt_ref, yt_ref, a_ref, b_ref, vkyw_ref, vkyb_ref, wkxw_ref,
          wkxb_ref, vkxw_ref, vkxb_ref, sig_ref, out_ref, u_ref, vky_ref):
    i = pl.program_id(0)
    bm = a_ref.shape[0]

    @pl.when(i == 0)
    def _prologue():
        cw = vkxw_ref[...] - 2.0 * wkxw_ref[...]
        cb = vkxb_ref[...] - 2.0 * wkxb_ref[...]
        u_ref[...] = (
            jax.lax.dot_general(
                xt_ref[...], cw,
                (((0,), (1,)), ((), ())),
                preferred_element_type=jnp.float32,
            )
            + cb[None, :]
        ).astype(jnp.bfloat16)
        vky_ref[...] = (
            jnp.dot(vkyw_ref[...], yt_ref[...],
                    preferred_element_type=jnp.float32)
            + vkyb_ref[...][:, None]
        )

    tt = jax.lax.dot_general(
        u_ref[...],
        a_ref[...].astype(jnp.bfloat16),
        (((0,), (1,)), ((), ())),
        preferred_element_type=jnp.float32,
    )
    vky_sl = vky_ref[:, pl.ds(i * bm, bm)]
    b_sl = b_ref[pl.ds(i * bm, bm)]
    out_ref[...] = jnp.maximum(
        vky_sl - sig_ref[0] * (b_sl[None, :] + tt), 0.0
    )


@functools.partial(jax.jit, static_argnames=())
def kernel(x, y, A, b, Vky_W, Vky_b, Wkx_W, Wkx_b, Vkx_W, Vkx_b, sigma):
    n, d = x.shape
    bm = 512
    grid = (n // bm,)

    full = lambda shape: pl.BlockSpec(shape, lambda i: (0, 0))
    anyb = lambda: pl.BlockSpec(memory_space=pltpu.VMEM)

    out_t = pl.pallas_call(
        _body,
        grid=grid,
        in_specs=[
            full((d, n)),
            full((d, n)),
            pl.BlockSpec((bm, n), lambda i: (i, 0)),
            anyb(),
            full((d, d)),
            anyb(),
            full((d, d)),
            anyb(),
            full((d, d)),
            anyb(),
            pl.BlockSpec(memory_space=pltpu.SMEM),
        ],
        out_specs=pl.BlockSpec((d, bm), lambda i: (0, i)),
        out_shape=jax.ShapeDtypeStruct((d, n), jnp.float32),
        scratch_shapes=[
            pltpu.VMEM((n, d), jnp.bfloat16),
            pltpu.VMEM((d, n), jnp.float32),
        ],
    )(
        x.T, y.T, A, b.reshape(n),
        Vky_W, Vky_b,
        Wkx_W, Wkx_b,
        Vkx_W, Vkx_b,
        sigma,
    )
    return out_t.T
